# Initial kernel scaffold; baseline (speedup 1.0000x reference)
import functools

import jax
import jax.numpy as jnp
from jax import lax
from jax.experimental import pallas as pl
from jax.experimental.pallas import tpu as pltpu
from jax.experimental.pallas import tpu_sc as plsc

_VOCAB = 1000000
_D = 64
_BATCH = 16384
_SEQ = 50
_N = _BATCH * _SEQ

_NC = 2
_NS = 16
_NW = _NC * _NS

_PER_W = _N // _NW
_CHUNK = 128
_NCHUNK = _PER_W // _CHUNK


def _emb_body(idx_hbm, table_hbm, out_hbm, idx_v, rows_v, gsem, ssem):
    wid = lax.axis_index("s") * _NC + lax.axis_index("c")
    pltpu.sync_copy(idx_hbm.at[wid], idx_v)

    def body(j, _):
        pltpu.async_copy(table_hbm.at[idx_v.at[j]], rows_v, gsem).wait()
        pltpu.async_copy(rows_v, out_hbm.at[wid, pl.ds(j * _CHUNK, _CHUNK)],
                         ssem).wait()
        return 0

    lax.fori_loop(0, _NCHUNK, body, 0)


_mesh = plsc.VectorSubcoreMesh(
    core_axis_name="c", subcore_axis_name="s",
    num_cores=_NC, num_subcores=_NS)

_emb = functools.partial(
    pl.kernel,
    out_type=jax.ShapeDtypeStruct((_NW, _PER_W, _D), jnp.float32),
    mesh=_mesh,
    scratch_types=[
        pltpu.VMEM((_NCHUNK, _CHUNK), jnp.int32),
        pltpu.VMEM((_CHUNK, _D), jnp.float32),
        pltpu.SemaphoreType.DMA,
        pltpu.SemaphoreType.DMA,
    ],
)(_emb_body)


@jax.jit
def kernel(x, embed_weight):
    idx = x.reshape(_NW, _NCHUNK, _CHUNK).astype(jnp.int32)
    out = _emb(idx, embed_weight)
    return out.reshape(_BATCH, _SEQ, _D)


# speedup vs baseline: 1.6869x; 1.6869x over previous
import functools

import jax
import jax.numpy as jnp
from jax import lax
from jax.experimental import pallas as pl
from jax.experimental.pallas import tpu as pltpu
from jax.experimental.pallas import tpu_sc as plsc

_VOCAB = 1000000
_D = 64
_BATCH = 16384
_SEQ = 50
_N = _BATCH * _SEQ

_NC = 2
_NS = 16
_NW = _NC * _NS

_PER_W = _N // _NW
_CHUNK = 128
_NCHUNK = _PER_W // _CHUNK


def _emb_body(idx_hbm, table_hbm, out_hbm, idx_v, rows_v, gsem, ssem):
    wid = lax.axis_index("s") * _NC + lax.axis_index("c")
    pltpu.sync_copy(idx_hbm.at[wid], idx_v)

    def body(j, _):
        pltpu.async_copy(table_hbm.at[idx_v.at[j]], rows_v, gsem).wait()
        pltpu.async_copy(rows_v, out_hbm.at[wid, pl.ds(j * _CHUNK, _CHUNK)],
                         ssem).wait()
        return 0

    lax.fori_loop(0, _NCHUNK, body, 0)


_mesh = plsc.VectorSubcoreMesh(
    core_axis_name="c", subcore_axis_name="s",
    num_cores=_NC, num_subcores=_NS)

_emb = functools.partial(
    pl.kernel,
    out_type=jax.ShapeDtypeStruct((_NW, _PER_W, _D), jnp.float32),
    mesh=_mesh,
    scratch_types=[
        pltpu.VMEM((_NCHUNK, _CHUNK), jnp.int32),
        pltpu.VMEM((_CHUNK, _D), jnp.float32),
        pltpu.SemaphoreType.DMA,
        pltpu.SemaphoreType.DMA,
    ],
    compiler_params=pltpu.CompilerParams(use_tc_tiling_on_sc=False),
)(_emb_body)


@jax.jit
def kernel(x, embed_weight):
    idx = x.reshape(_NW, _NCHUNK, _CHUNK).astype(jnp.int32)
    out = _emb(idx, embed_weight)
    return out.reshape(_BATCH, _SEQ, _D)


# speedup vs baseline: 1.8733x; 1.1105x over previous
import functools

import jax
import jax.numpy as jnp
from jax import lax
from jax.experimental import pallas as pl
from jax.experimental.pallas import tpu as pltpu
from jax.experimental.pallas import tpu_sc as plsc

_VOCAB = 1000000
_D = 64
_BATCH = 16384
_SEQ = 50
_N = _BATCH * _SEQ

_NC = 2
_NS = 16
_NW = _NC * _NS

_PER_W = _N // _NW
_CHUNK = 128
_NCHUNK = _PER_W // _CHUNK


_K = 4
_NGROUP = _NCHUNK // _K


def _emb_body(idx_hbm, table_hbm, out_hbm, idx_v, rows_v, gsem, ssem):
    wid = lax.axis_index("s") * _NC + lax.axis_index("c")
    pltpu.sync_copy(idx_hbm.at[wid], idx_v)

    def fire_gathers(g, h):
        for b in range(_K):
            pltpu.async_copy(table_hbm.at[idx_v.at[g * _K + b]],
                             rows_v.at[h, b], gsem)

    fire_gathers(0, 0)

    def body(g, _):
        h = lax.rem(g, 2)
        for b in range(_K):
            pltpu.make_async_copy(table_hbm.at[pl.ds(0, _CHUNK)],
                                  rows_v.at[h, b], gsem).wait()
        @pl.when(g + 1 < _NGROUP)
        def _():
            fire_gathers(g + 1, 1 - h)
        for b in range(_K):
            pltpu.async_copy(
                rows_v.at[h, b],
                out_hbm.at[wid, pl.ds((g * _K + b) * _CHUNK, _CHUNK)], ssem)
        for b in range(_K):
            pltpu.make_async_copy(
                rows_v.at[h, b], out_hbm.at[wid, pl.ds(0, _CHUNK)],
                ssem).wait()
        return 0

    lax.fori_loop(0, _NGROUP, body, 0)


_mesh = plsc.VectorSubcoreMesh(
    core_axis_name="c", subcore_axis_name="s",
    num_cores=_NC, num_subcores=_NS)

_emb = functools.partial(
    pl.kernel,
    out_type=jax.ShapeDtypeStruct((_NW, _PER_W, _D), jnp.float32),
    mesh=_mesh,
    scratch_types=[
        pltpu.VMEM((_NCHUNK, _CHUNK), jnp.int32),
        pltpu.VMEM((2, _K, _CHUNK, _D), jnp.float32),
        pltpu.SemaphoreType.DMA,
        pltpu.SemaphoreType.DMA,
    ],
    compiler_params=pltpu.CompilerParams(use_tc_tiling_on_sc=False),
)(_emb_body)


@jax.jit
def kernel(x, embed_weight):
    idx = x.reshape(_NW, _NCHUNK, _CHUNK).astype(jnp.int32)
    out = _emb(idx, embed_weight)
    return out.reshape(_BATCH, _SEQ, _D)
